# Initial kernel scaffold; baseline (speedup 1.0000x reference)
import functools

import jax
import jax.numpy as jnp
from jax import lax
from jax.experimental import pallas as pl
from jax.experimental.pallas import tpu as pltpu
from jax.experimental.pallas import tpu_sc as plsc

_NC = 2
_NS = 16
_NW = _NC * _NS
_C = 128


@functools.lru_cache(maxsize=None)
def _build_sc_scatter(N, D, NCHUNK, NPAD):
    ZROWS = NPAD // _NS

    mesh = plsc.VectorSubcoreMesh(core_axis_name="c", subcore_axis_name="s")

    @functools.partial(
        pl.kernel,
        out_type=jax.ShapeDtypeStruct((_NC, NPAD, D), jnp.float32),
        mesh=mesh,
        scratch_types=[
            pltpu.VMEM((NCHUNK, _C), jnp.int32),
            pltpu.VMEM((NCHUNK, _C), jnp.int32),
            pltpu.VMEM((_C, D), jnp.float32),
            pltpu.VMEM_SHARED((NPAD, D), jnp.float32),
            pltpu.SemaphoreType.DMA,
        ],
    )
    def sc_scatter(x_hbm, src_hbm, dst_hbm, zero_hbm, out_hbm,
                   src_v, dst_v, mbuf, acc, sem):
        cid = lax.axis_index("c")
        sid = lax.axis_index("s")
        wid = sid * _NC + cid

        pltpu.sync_copy(src_hbm.at[wid], src_v)
        pltpu.sync_copy(dst_hbm.at[wid], dst_v)

        pltpu.sync_copy(zero_hbm, acc.at[pl.ds(sid * ZROWS, ZROWS)])
        plsc.subcore_barrier()

        def chunk(j, carry):
            pltpu.async_copy(x_hbm.at[src_v.at[j]], mbuf, sem).wait()
            pltpu.sync_copy(mbuf, acc.at[dst_v.at[j]], add=True)
            return carry

        lax.fori_loop(0, NCHUNK, chunk, 0)
        plsc.subcore_barrier()

        pltpu.sync_copy(acc.at[pl.ds(sid * ZROWS, ZROWS)],
                        out_hbm.at[cid, pl.ds(sid * ZROWS, ZROWS)])

    return sc_scatter


def _add_partials(partial, N, D):
    def body(a_ref, b_ref, o_ref):
        o_ref[...] = a_ref[0] + b_ref[0]

    BR = 2000
    return pl.pallas_call(
        body,
        out_shape=jax.ShapeDtypeStruct((N, D), jnp.float32),
        grid=(N // BR,),
        in_specs=[
            pl.BlockSpec((1, BR, D), lambda i: (0, i, 0)),
            pl.BlockSpec((1, BR, D), lambda i: (1, i, 0)),
        ],
        out_specs=pl.BlockSpec((BR, D), lambda i: (i, 0)),
    )(partial, partial)


def kernel(x, edge_index):
    N, D = x.shape
    E = edge_index.shape[1]

    NCHUNK = -(-E // (_NW * _C))
    EPAD = _NW * NCHUNK * _C
    NPAD = -(-(N + 1) // (_NS * 8)) * (_NS * 8)

    src = edge_index[0].astype(jnp.int32)
    dst = edge_index[1].astype(jnp.int32)
    pad = EPAD - E
    if pad:
        src = jnp.concatenate([src, jnp.zeros((pad,), jnp.int32)])
        dst = jnp.concatenate([dst, jnp.full((pad,), N, jnp.int32)])
    src_r = src.reshape(_NW, NCHUNK, _C)
    dst_r = dst.reshape(_NW, NCHUNK, _C)
    zeros = jnp.zeros((NPAD // _NS, D), jnp.float32)

    partial = _build_sc_scatter(N, D, NCHUNK, NPAD)(x, src_r, dst_r, zeros)
    return _add_partials(partial, N, D)


# speedup vs baseline: 10.9422x; 10.9422x over previous
import functools

import jax
import jax.numpy as jnp
from jax import lax
from jax.experimental import pallas as pl
from jax.experimental.pallas import tpu as pltpu
from jax.experimental.pallas import tpu_sc as plsc

_NC = 2
_NS = 16
_C = 128
_W = 16
_NB = 4


@functools.lru_cache(maxsize=None)
def _build_sc_segsum(N, D, NCHUNK, NPAD):
    DH = D // _NC
    ZROWS = NPAD // _NS
    CROWS = NPAD // _NS
    NWINT = NCHUNK // _W

    mesh = plsc.VectorSubcoreMesh(core_axis_name="c", subcore_axis_name="s")

    @functools.partial(
        pl.kernel,
        out_type=(jax.ShapeDtypeStruct((N, DH), jnp.float32),
                  jax.ShapeDtypeStruct((N, DH), jnp.float32)),
        mesh=mesh,
        scratch_types=[
            pltpu.VMEM((2, _W, _C), jnp.int32),
            pltpu.VMEM((2, _W, _C), jnp.int32),
            [pltpu.VMEM((_C, DH), jnp.float32) for _ in range(_NB)],
            pltpu.VMEM_SHARED((N, DH), jnp.float32),
            pltpu.VMEM_SHARED((NPAD, DH), jnp.float32),
            [pltpu.SemaphoreType.DMA for _ in range(_NB)],
            [pltpu.SemaphoreType.DMA for _ in range(_NB)],
            pltpu.SemaphoreType.DMA,
        ],
        compiler_params=pltpu.CompilerParams(use_tc_tiling_on_sc=False),
    )
    def sc_segsum(x0_hbm, x1_hbm, src_hbm, dst_hbm, zero_hbm,
                  o0_hbm, o1_hbm,
                  src_w, dst_w, mb, x_sp, acc, gsem, ssem, semi):
        cid = lax.axis_index("c")
        sid = lax.axis_index("s")

        def rows_split(fn):
            @pl.when(sid < _NS - 1)
            def _():
                fn(pl.ds(sid * CROWS, CROWS))

            @pl.when(sid == _NS - 1)
            def _():
                fn(pl.ds((_NS - 1) * CROWS, N - (_NS - 1) * CROWS))

        def refill(win, slot):
            pltpu.async_copy(
                src_hbm.at[sid, pl.ds(win * _W, _W)], src_w.at[slot], semi)
            pltpu.async_copy(
                dst_hbm.at[sid, pl.ds(win * _W, _W)], dst_w.at[slot], semi)

        def refill_wait(win, slot):
            pltpu.make_async_copy(
                src_hbm.at[sid, pl.ds(win * _W, _W)], src_w.at[slot],
                semi).wait()
            pltpu.make_async_copy(
                dst_hbm.at[sid, pl.ds(win * _W, _W)], dst_w.at[slot],
                semi).wait()

        def gather_start(slot, k):
            b = k % _NB
            pltpu.async_copy(x_sp.at[src_w.at[slot, k]], mb[b], gsem[b])

        def gather_wait(slot, k):
            b = k % _NB
            pltpu.make_async_copy(
                x_sp.at[src_w.at[slot, k]], mb[b], gsem[b]).wait()

        def scatter_start(slot, k):
            b = k % _NB
            pltpu.async_copy(mb[b], acc.at[dst_w.at[slot, k]], ssem[b],
                             add=True)

        def scatter_wait(slot, k):
            b = k % _NB
            pltpu.make_async_copy(
                mb[b], acc.at[dst_w.at[slot, k]], ssem[b]).wait()

        def window(w, slot, nslot, first, last):
            for k in range(_W):
                gather_wait(slot, k)
                scatter_start(slot, k)
                if k == 0 and not last:
                    refill(w + 1, nslot)
                if not (first and k < 2):
                    if k < 2:
                        scatter_wait(nslot, _W - 2 + k)
                    else:
                        scatter_wait(slot, k - 2)
                if last and k >= _W - 2:
                    continue
                if k == _W - 2 and not last:
                    refill_wait(w + 1, nslot)
                if k < _W - 2:
                    gather_start(slot, k + 2)
                else:
                    gather_start(nslot, k - (_W - 2))

        @pl.when(cid == 0)
        def _():
            rows_split(lambda r: pltpu.sync_copy(x0_hbm.at[r], x_sp.at[r]))

        @pl.when(cid == 1)
        def _():
            rows_split(lambda r: pltpu.sync_copy(x1_hbm.at[r], x_sp.at[r]))

        pltpu.sync_copy(src_hbm.at[sid, pl.ds(0, _W)], src_w.at[0])
        pltpu.sync_copy(dst_hbm.at[sid, pl.ds(0, _W)], dst_w.at[0])
        pltpu.sync_copy(zero_hbm, acc.at[pl.ds(sid * ZROWS, ZROWS)])
        plsc.subcore_barrier()

        gather_start(0, 0)
        gather_start(0, 1)

        window(0, 0, 1, first=True, last=False)

        def mid(w, carry):
            slot = lax.rem(w, 2)
            window(w, slot, 1 - slot, first=False, last=False)
            return carry

        lax.fori_loop(1, NWINT - 1, mid, 0)

        wl = NWINT - 1
        window(wl, wl % 2, (wl + 1) % 2, first=False, last=True)

        for k in range(_W - 2, _W):
            scatter_wait(wl % 2, k)
        plsc.subcore_barrier()

        @pl.when(cid == 0)
        def _():
            rows_split(lambda r: pltpu.sync_copy(acc.at[r], o0_hbm.at[r]))

        @pl.when(cid == 1)
        def _():
            rows_split(lambda r: pltpu.sync_copy(acc.at[r], o1_hbm.at[r]))

    return sc_segsum


def kernel(x, edge_index):
    N, D = x.shape
    E = edge_index.shape[1]

    NCHUNK = -(-E // (_NS * _C))
    NCHUNK = -(-NCHUNK // _W) * _W
    EPAD = _NS * NCHUNK * _C
    NPAD = -(-(N + 1) // (_NS * 8)) * (_NS * 8)

    src = edge_index[0].astype(jnp.int32)
    dst = edge_index[1].astype(jnp.int32)
    pad = EPAD - E
    if pad:
        src = jnp.concatenate([src, jnp.zeros((pad,), jnp.int32)])
        dst = jnp.concatenate([dst, jnp.full((pad,), N, jnp.int32)])
    src_r = src.reshape(_NS, NCHUNK, _C)
    dst_r = dst.reshape(_NS, NCHUNK, _C)
    zeros = jnp.zeros((NPAD // _NS, D // _NC), jnp.float32)

    DH = D // _NC
    out0, out1 = _build_sc_segsum(N, D, NCHUNK, NPAD)(
        x[:, :DH], x[:, DH:], src_r, dst_r, zeros)
    return jnp.concatenate([out0, out1], axis=1)


# speedup vs baseline: 12.8115x; 1.1708x over previous
import functools

import jax
import jax.numpy as jnp
from jax import lax
from jax.experimental import pallas as pl
from jax.experimental.pallas import tpu as pltpu
from jax.experimental.pallas import tpu_sc as plsc

_NC = 2
_NS = 16
_C = 128
_W = 16
_NB = 4


@functools.lru_cache(maxsize=None)
def _build_sc_segsum(N, D, NCHUNK, NPAD):
    DH = D // _NC
    ZROWS = NPAD // _NS
    CROWS = NPAD // _NS
    NWINT = NCHUNK // _W

    mesh = plsc.VectorSubcoreMesh(core_axis_name="c", subcore_axis_name="s")

    @functools.partial(
        pl.kernel,
        out_type=jax.ShapeDtypeStruct((N, D), jnp.float32),
        mesh=mesh,
        scratch_types=[
            pltpu.VMEM((2, _W, _C), jnp.int32),
            pltpu.VMEM((2, _W, _C), jnp.int32),
            [pltpu.VMEM((_C, DH), jnp.float32) for _ in range(_NB)],
            pltpu.VMEM_SHARED((N, DH), jnp.float32),
            pltpu.VMEM_SHARED((NPAD, DH), jnp.float32),
            [pltpu.SemaphoreType.DMA for _ in range(_NB)],
            [pltpu.SemaphoreType.DMA for _ in range(_NB)],
            pltpu.SemaphoreType.DMA,
        ],
        compiler_params=pltpu.CompilerParams(use_tc_tiling_on_sc=False),
    )
    def sc_segsum(x_hbm, src_hbm, dst_hbm, out_hbm,
                  src_w, dst_w, mb, x_sp, acc, gsem, ssem, semi):
        cid = lax.axis_index("c")
        sid = lax.axis_index("s")
        col0 = cid * DH

        def rows_split(fn):
            @pl.when(sid < _NS - 1)
            def _():
                fn(pl.ds(sid * CROWS, CROWS))

            @pl.when(sid == _NS - 1)
            def _():
                fn(pl.ds((_NS - 1) * CROWS, N - (_NS - 1) * CROWS))

        def refill(win, slot):
            pltpu.async_copy(
                src_hbm.at[sid, pl.ds(win * _W, _W)], src_w.at[slot], semi)
            pltpu.async_copy(
                dst_hbm.at[sid, pl.ds(win * _W, _W)], dst_w.at[slot], semi)

        def refill_wait(win, slot):
            pltpu.make_async_copy(
                src_hbm.at[sid, pl.ds(win * _W, _W)], src_w.at[slot],
                semi).wait()
            pltpu.make_async_copy(
                dst_hbm.at[sid, pl.ds(win * _W, _W)], dst_w.at[slot],
                semi).wait()

        def gather_start(slot, k):
            b = k % _NB
            pltpu.async_copy(x_sp.at[src_w.at[slot, k]], mb[b], gsem[b])

        def gather_wait(slot, k):
            b = k % _NB
            pltpu.make_async_copy(
                x_sp.at[src_w.at[slot, k]], mb[b], gsem[b]).wait()

        def scatter_start(slot, k):
            b = k % _NB
            pltpu.async_copy(mb[b], acc.at[dst_w.at[slot, k]], ssem[b],
                             add=True)

        def scatter_wait(slot, k):
            b = k % _NB
            pltpu.make_async_copy(
                mb[b], acc.at[dst_w.at[slot, k]], ssem[b]).wait()

        def window(w, slot, nslot, first, last):
            for k in range(_W):
                gather_wait(slot, k)
                scatter_start(slot, k)
                if k == 0 and not last:
                    refill(w + 1, nslot)
                if not (first and k < 2):
                    if k < 2:
                        scatter_wait(nslot, _W - 2 + k)
                    else:
                        scatter_wait(slot, k - 2)
                if last and k >= _W - 2:
                    continue
                if k == _W - 2 and not last:
                    refill_wait(w + 1, nslot)
                if k < _W - 2:
                    gather_start(slot, k + 2)
                else:
                    gather_start(nslot, k - (_W - 2))

        rows_split(lambda r: pltpu.sync_copy(
            x_hbm.at[r, pl.ds(col0, DH)], x_sp.at[r]))
        pltpu.sync_copy(src_hbm.at[sid, pl.ds(0, _W)], src_w.at[0])
        pltpu.sync_copy(dst_hbm.at[sid, pl.ds(0, _W)], dst_w.at[0])

        z16 = jnp.zeros((16,), jnp.float32)

        def zrow(r, carry):
            for c in range(DH // 16):
                mb[0][r, pl.ds(c * 16, 16)] = z16
            return carry

        lax.fori_loop(0, _C, zrow, 0)
        zbase = sid * ZROWS
        full, rem = divmod(ZROWS, _C)
        for i in range(full):
            pltpu.sync_copy(mb[0], acc.at[pl.ds(zbase + i * _C, _C)])
        if rem:
            pltpu.sync_copy(mb[0].at[pl.ds(0, rem)],
                            acc.at[pl.ds(zbase + full * _C, rem)])
        plsc.subcore_barrier()

        gather_start(0, 0)
        gather_start(0, 1)

        window(0, 0, 1, first=True, last=False)

        def mid(w, carry):
            slot = lax.rem(w, 2)
            window(w, slot, 1 - slot, first=False, last=False)
            return carry

        lax.fori_loop(1, NWINT - 1, mid, 0)

        wl = NWINT - 1
        window(wl, wl % 2, (wl + 1) % 2, first=False, last=True)

        for k in range(_W - 2, _W):
            scatter_wait(wl % 2, k)
        plsc.subcore_barrier()

        rows_split(lambda r: pltpu.sync_copy(
            acc.at[r], out_hbm.at[r, pl.ds(col0, DH)]))

    return sc_segsum


def kernel(x, edge_index):
    N, D = x.shape
    E = edge_index.shape[1]

    NCHUNK = -(-E // (_NS * _C))
    NCHUNK = -(-NCHUNK // _W) * _W
    EPAD = _NS * NCHUNK * _C
    NPAD = -(-(N + 1) // (_NS * 8)) * (_NS * 8)

    src = edge_index[0].astype(jnp.int32)
    dst = edge_index[1].astype(jnp.int32)
    pad = EPAD - E
    if pad:
        src = jnp.concatenate([src, jnp.zeros((pad,), jnp.int32)])
        dst = jnp.concatenate([dst, jnp.full((pad,), N, jnp.int32)])
    src_r = src.reshape(_NS, NCHUNK, _C)
    dst_r = dst.reshape(_NS, NCHUNK, _C)

    return _build_sc_segsum(N, D, NCHUNK, NPAD)(x, src_r, dst_r)
